# Initial kernel scaffold; baseline (speedup 1.0000x reference)
import jax
import jax.numpy as jnp
from jax.experimental import pallas as pl


def kernel(index, length, data):
    raise NotImplementedError("write your pallas kernel here")


# speedup vs baseline: 4.1511x; 4.1511x over previous
import functools

import jax
import jax.numpy as jnp
from jax import lax
from jax.experimental import pallas as pl
from jax.experimental.pallas import tpu as pltpu
from jax.experimental.pallas import tpu_sc as plsc

L = 200
C = 64


def _sc_block_gather(starts, table_flat, batch):
    num_workers = 32
    per_w = batch // num_workers
    blk = L * C
    nbuf = 8
    mesh = plsc.VectorSubcoreMesh(core_axis_name="c", subcore_axis_name="s")

    @functools.partial(
        pl.kernel,
        out_type=jax.ShapeDtypeStruct((batch * blk,), jnp.float32),
        mesh=mesh,
        scratch_types=[
            pltpu.VMEM((per_w,), jnp.int32),
            [pltpu.VMEM((blk,), jnp.float32) for _ in range(nbuf)],
            pltpu.SemaphoreType.DMA((nbuf,)),
            pltpu.SemaphoreType.DMA((nbuf,)),
        ],
    )
    def k(starts_hbm, table_hbm, out_hbm, idx_v, bufs, in_sems, out_sems):
        wid = lax.axis_index("s") * 2 + lax.axis_index("c")
        base = wid * per_w
        pltpu.sync_copy(starts_hbm.at[pl.ds(base, per_w)], idx_v)
        svecs = [idx_v[pl.ds(g * 16, 16)] for g in range(per_w // 16)]

        def gather(i):
            p = i % nbuf
            sj = svecs[i // 16][i % 16]
            return pltpu.make_async_copy(
                table_hbm.at[pl.ds(sj * C, blk)], bufs[p], in_sems.at[p]
            )

        def scatter(i):
            p = i % nbuf
            return pltpu.make_async_copy(
                bufs[p], out_hbm.at[pl.ds((base + i) * blk, blk)], out_sems.at[p]
            )

        in_h = {}
        out_h = {}
        for i in range(min(nbuf, per_w)):
            in_h[i] = gather(i)
            in_h[i].start()
        for i in range(per_w):
            in_h[i].wait()
            out_h[i] = scatter(i)
            out_h[i].start()
            nxt = i + nbuf
            if nxt < per_w:
                out_h[i].wait()
                in_h[nxt] = gather(nxt)
                in_h[nxt].start()
        for i in range(max(0, per_w - nbuf), per_w):
            out_h[i].wait()

    return k(starts, table_flat)


def kernel(index, length, data):
    p = data.shape[0]
    batch = index.shape[0]
    starts = jnp.mod(index + (jnp.asarray(length, index.dtype) - L), p)
    table_ext = jnp.concatenate([data, data[: L - 1]], axis=0)
    out_flat = _sc_block_gather(
        starts.astype(jnp.int32), table_ext.reshape(-1), batch
    )
    return out_flat.reshape(batch, L, C)
